# baseline (device time: 95106 ns/iter reference)
import jax
import jax.numpy as jnp
from jax import lax
from jax.experimental import pallas as pl
from jax.experimental.pallas import tpu as pltpu

N_DEV = 32
B = 2
SQ = 256
D_MODEL = 768
H_LOC = 8
GQA_GROUP = 4
KV_LOC = H_LOC // GQA_GROUP
DH = 64
SKV = 512
ROWS = B * SQ
N_CHUNKS = N_DEV
CHUNK_ROWS = ROWS // N_CHUNKS
CHUNKS_PER_B = SQ // CHUNK_ROWS

BARRIER_MASKS = (1, 2, 3, 4, 8, 12, 16)


def _body(x_ref, wq_ref, wo_ref, k_hbm_ref, v_hbm_ref, out_ref,
          acc_ref, rs_send_ref, rs_recv_ref, ag_send_ref, ag_recv_ref,
          k_ref, v_ref, send_sems, rs_sems, ag_sems, kv_sems):
    my = lax.axis_index("i")
    qa = my & 3
    qb = (my >> 2) & 3
    b_keep = (my >> 1) & 1

    kv_copies = []
    for b in range(B):
        for g in range(KV_LOC):
            h_idx = KV_LOC * my + g
            for src, dst, si in ((k_hbm_ref, k_ref, 0), (v_hbm_ref, v_ref, 4)):
                cp = pltpu.make_async_copy(
                    src.at[b, :, pl.ds(h_idx, 1), :],
                    dst.at[b, :, pl.ds(g, 1), :],
                    kv_sems.at[si + b * KV_LOC + g],
                )
                cp.start()
                kv_copies.append(cp)

    barrier = pltpu.get_barrier_semaphore()
    for msk in BARRIER_MASKS:
        pl.semaphore_signal(barrier, inc=1, device_id=(my ^ msk,),
                            device_id_type=pl.DeviceIdType.MESH)
    pl.semaphore_wait(barrier, len(BARRIER_MASKS))

    for cp in kv_copies:
        cp.wait()

    def compute_partial(b):
        qmat = jnp.dot(x_ref[b], wq_ref[...],
                       preferred_element_type=jnp.float32)
        outs = []
        for h in range(H_LOC):
            q = qmat[:, h * DH:(h + 1) * DH]
            k = k_ref[b, :, h // GQA_GROUP, :]
            v = v_ref[b, :, h // GQA_GROUP, :]
            s = jnp.dot(q, k.T, preferred_element_type=jnp.float32) * 0.125
            m = jnp.max(s, axis=1, keepdims=True)
            p = jnp.exp(s - m)
            l = jnp.sum(p, axis=1, keepdims=True)
            outs.append(jnp.dot(p, v, preferred_element_type=jnp.float32) / l)
        o = jnp.concatenate(outs, axis=1)
        pb = jnp.dot(o, wo_ref[...],
                     preferred_element_type=jnp.float32)
        acc_ref[pl.ds(b * CHUNKS_PER_B, CHUNKS_PER_B)] = pb.reshape(
            CHUNKS_PER_B, CHUNK_ROWS, D_MODEL)

    def exchange(send_buf, send_ofs, recv_buf, recv_ofs, n, sem_i, recv_sem,
                 partner):
        rdma = pltpu.make_async_remote_copy(
            src_ref=send_buf.at[pl.ds(send_ofs, n)],
            dst_ref=recv_buf.at[pl.ds(recv_ofs, n)],
            send_sem=send_sems.at[sem_i],
            recv_sem=recv_sem,
            device_id=(partner,),
            device_id_type=pl.DeviceIdType.MESH,
        )
        rdma.start()
        return rdma

    @pl.when(b_keep == 0)
    def _():
        compute_partial(1)

    @pl.when(b_keep == 1)
    def _():
        compute_partial(0)

    rdma_a = {}
    for d in (2, 3):
        ofs = (d - 1) * 8
        rs_send_ref[pl.ds(ofs, 8)] = acc_ref[
            pl.ds((qa ^ d) * 8, 8)].astype(jnp.bfloat16)
        rdma_a[d] = exchange(rs_send_ref, ofs, rs_recv_ref, ofs, 8,
                             d - 1, rs_sems.at[d - 1], my ^ d)

    @pl.when(b_keep == 0)
    def _():
        compute_partial(0)

    @pl.when(b_keep == 1)
    def _():
        compute_partial(1)

    rs_send_ref[pl.ds(0, 8)] = acc_ref[pl.ds((qa ^ 1) * 8, 8)].astype(
        jnp.bfloat16)
    rdma_a[1] = exchange(rs_send_ref, 0, rs_recv_ref, 0, 8,
                         0, rs_sems.at[0], my ^ 1)

    for d in (1, 3, 2):
        rdma_a[d].wait_recv()
    acc_ref[pl.ds(qa * 8, 8)] = (
        acc_ref[pl.ds(qa * 8, 8)]
        + rs_recv_ref[pl.ds(0, 8)].astype(jnp.float32)
        + rs_recv_ref[pl.ds(8, 8)].astype(jnp.float32)
        + rs_recv_ref[pl.ds(16, 8)].astype(jnp.float32))
    for d in (1, 2, 3):
        rdma_a[d].wait_send()

    rb = qa * 8
    rdma_b = {}
    for d in (1, 2, 3):
        ofs = 24 + (d - 1) * 2
        rs_send_ref[pl.ds(ofs, 2)] = acc_ref[
            pl.ds(rb + (qb ^ d) * 2, 2)].astype(jnp.bfloat16)
        rdma_b[d] = exchange(rs_send_ref, ofs, rs_recv_ref, ofs, 2,
                             d - 1, rs_sems.at[3 + d - 1], my ^ (d << 2))
    for d in (1, 2, 3):
        rdma_b[d].wait_recv()
    fs = rb + qb * 2
    acc_ref[pl.ds(fs, 2)] = (
        acc_ref[pl.ds(fs, 2)]
        + rs_recv_ref[pl.ds(24, 2)].astype(jnp.float32)
        + rs_recv_ref[pl.ds(26, 2)].astype(jnp.float32)
        + rs_recv_ref[pl.ds(28, 2)].astype(jnp.float32))
    for d in (1, 2, 3):
        rdma_b[d].wait_send()

    rs_send_ref[pl.ds(30, 2)] = acc_ref[pl.ds(fs, 2)].astype(jnp.bfloat16)
    are = exchange(rs_send_ref, 30, rs_recv_ref, 30, 2,
                   0, rs_sems.at[6], my ^ 16)
    are.wait_recv()
    acc_ref[pl.ds(fs, 2)] = (
        acc_ref[pl.ds(fs, 2)]
        + rs_recv_ref[pl.ds(30, 2)].astype(jnp.float32))
    are.wait_send()

    ag_send_ref[pl.ds(0, 2)] = acc_ref[pl.ds(fs, 2)].astype(jnp.bfloat16)
    rdma_gb = {}
    for d in (1, 2, 3):
        rdma_gb[d] = exchange(ag_send_ref, 0, ag_recv_ref, (d - 1) * 2, 2,
                              d - 1, ag_sems.at[d - 1], my ^ (d << 2))
    for d in (1, 2, 3):
        rdma_gb[d].wait_recv()
        acc_ref[pl.ds(rb + (qb ^ d) * 2, 2)] = ag_recv_ref[
            pl.ds((d - 1) * 2, 2)].astype(jnp.float32)
    for d in (1, 2, 3):
        rdma_gb[d].wait_send()

    ag_send_ref[pl.ds(2, 8)] = acc_ref[pl.ds(rb, 8)].astype(jnp.bfloat16)
    rdma_ga = {}
    for d in (1, 2, 3):
        rdma_ga[d] = exchange(ag_send_ref, 2, ag_recv_ref, 6 + (d - 1) * 8, 8,
                              d - 1, ag_sems.at[3 + d - 1], my ^ d)
    for d in (1, 2, 3):
        rdma_ga[d].wait_recv()
        acc_ref[pl.ds((qa ^ d) * 8, 8)] = ag_recv_ref[
            pl.ds(6 + (d - 1) * 8, 8)].astype(jnp.float32)
    for d in (1, 2, 3):
        rdma_ga[d].wait_send()

    for b in range(B):
        out_ref[b] = acc_ref[b * CHUNKS_PER_B:(b + 1) * CHUNKS_PER_B].reshape(
            SQ, D_MODEL)


def kernel(x, Wq, Wo, K_ext, V_ext):
    return pl.pallas_call(
        _body,
        out_shape=jax.ShapeDtypeStruct((B, SQ, D_MODEL), jnp.float32),
        in_specs=[pl.BlockSpec(memory_space=pltpu.VMEM)] * 3
        + [pl.BlockSpec(memory_space=pl.ANY)] * 2,
        out_specs=pl.BlockSpec(memory_space=pltpu.VMEM),
        scratch_shapes=[
            pltpu.VMEM((N_CHUNKS, CHUNK_ROWS, D_MODEL), jnp.float32),
            pltpu.VMEM((32, CHUNK_ROWS, D_MODEL), jnp.bfloat16),
            pltpu.VMEM((32, CHUNK_ROWS, D_MODEL), jnp.bfloat16),
            pltpu.VMEM((10, CHUNK_ROWS, D_MODEL), jnp.bfloat16),
            pltpu.VMEM((30, CHUNK_ROWS, D_MODEL), jnp.bfloat16),
            pltpu.VMEM((B, SKV, KV_LOC, DH), jnp.float32),
            pltpu.VMEM((B, SKV, KV_LOC, DH), jnp.float32),
            pltpu.SemaphoreType.DMA((3,)),
            pltpu.SemaphoreType.DMA((7,)),
            pltpu.SemaphoreType.DMA((6,)),
            pltpu.SemaphoreType.DMA((8,)),
        ],
        compiler_params=pltpu.CompilerParams(collective_id=0),
    )(x, Wq, Wo, K_ext, V_ext)


# device time: 38015 ns/iter; 2.5018x vs baseline; 2.5018x over previous
import jax
import jax.numpy as jnp
from jax import lax
from jax.experimental import pallas as pl
from jax.experimental.pallas import tpu as pltpu

N_DEV = 32
B = 2
SQ = 256
D_MODEL = 768
H_LOC = 8
GQA_GROUP = 4
KV_LOC = H_LOC // GQA_GROUP
DH = 64
SKV = 512
ROWS = B * SQ
N_CHUNKS = N_DEV
CHUNK_ROWS = ROWS // N_CHUNKS
CHUNKS_PER_B = SQ // CHUNK_ROWS

BARRIER_MASKS = (1, 2, 3, 4, 8, 12, 16)


def _body(x_ref, wq_ref, wo_ref, k_ref, v_ref, out_ref,
          acc_ref, rs_send_ref, rs_recv_ref, ag_send_ref, ag_recv_ref,
          send_sems, rs_sems, ag_sems):
    my = lax.axis_index("i")
    qa = my & 3
    qb = (my >> 2) & 3
    b_keep = (my >> 1) & 1

    barrier = pltpu.get_barrier_semaphore()
    for msk in BARRIER_MASKS:
        pl.semaphore_signal(barrier, inc=1, device_id=(my ^ msk,),
                            device_id_type=pl.DeviceIdType.MESH)
    pl.semaphore_wait(barrier, len(BARRIER_MASKS))

    def compute_rows(b, r0, nrows):
        qmat = jnp.dot(x_ref[b, r0:r0 + nrows], wq_ref[...],
                       preferred_element_type=jnp.float32)
        outs = []
        for h in range(H_LOC):
            q = qmat[:, h * DH:(h + 1) * DH]
            k = k_ref[b, h // GQA_GROUP]
            v = v_ref[b, h // GQA_GROUP]
            s = jnp.dot(q, k.T, preferred_element_type=jnp.float32) * 0.125
            m = jnp.max(s, axis=1, keepdims=True)
            p = jnp.exp(s - m)
            l = jnp.sum(p, axis=1, keepdims=True)
            outs.append(jnp.dot(p, v, preferred_element_type=jnp.float32) / l)
        o = jnp.concatenate(outs, axis=1)
        pb = jnp.dot(o, wo_ref[...],
                     preferred_element_type=jnp.float32)
        acc_ref[pl.ds(b * CHUNKS_PER_B + r0 // CHUNK_ROWS,
                      nrows // CHUNK_ROWS)] = pb.reshape(
            nrows // CHUNK_ROWS, CHUNK_ROWS, D_MODEL)

    def compute_partial(b):
        compute_rows(b, 0, SQ)

    def exchange(send_buf, send_ofs, recv_buf, recv_ofs, n, sem_i, recv_sem,
                 partner):
        rdma = pltpu.make_async_remote_copy(
            src_ref=send_buf.at[pl.ds(send_ofs, n)],
            dst_ref=recv_buf.at[pl.ds(recv_ofs, n)],
            send_sem=send_sems.at[sem_i],
            recv_sem=recv_sem,
            device_id=(partner,),
            device_id_type=pl.DeviceIdType.MESH,
        )
        rdma.start()
        return rdma

    @pl.when(b_keep == 0)
    def _():
        compute_partial(1)

    @pl.when(b_keep == 1)
    def _():
        compute_partial(0)

    rdma_a = {}
    for d in (2, 3):
        ofs = (d - 1) * 8
        rs_send_ref[pl.ds(ofs, 8)] = acc_ref[
            pl.ds((qa ^ d) * 8, 8)].astype(jnp.bfloat16)
        rdma_a[d] = exchange(rs_send_ref, ofs, rs_recv_ref, ofs, 8,
                             d - 1, rs_sems.at[d - 1], my ^ d)

    for qv in range(4):
        @pl.when(qa == qv)
        def _(qv=qv):
            cq = qv ^ 1
            compute_rows(cq >> 1, (cq & 1) * 128, 128)

    rs_send_ref[pl.ds(0, 8)] = acc_ref[pl.ds((qa ^ 1) * 8, 8)].astype(
        jnp.bfloat16)
    rdma_a[1] = exchange(rs_send_ref, 0, rs_recv_ref, 0, 8,
                         0, rs_sems.at[0], my ^ 1)

    for qv in range(4):
        @pl.when(qa == qv)
        def _(qv=qv):
            compute_rows(qv >> 1, (qv & 1) * 128, 128)

    for d in (1, 3, 2):
        rdma_a[d].wait_recv()
    acc_ref[pl.ds(qa * 8, 8)] = (
        acc_ref[pl.ds(qa * 8, 8)]
        + rs_recv_ref[pl.ds(0, 8)].astype(jnp.float32)
        + rs_recv_ref[pl.ds(8, 8)].astype(jnp.float32)
        + rs_recv_ref[pl.ds(16, 8)].astype(jnp.float32))
    for d in (1, 2, 3):
        rdma_a[d].wait_send()

    rb = qa * 8
    rdma_b = {}
    for d in (1, 2, 3):
        ofs = 24 + (d - 1) * 2
        rs_send_ref[pl.ds(ofs, 2)] = acc_ref[
            pl.ds(rb + (qb ^ d) * 2, 2)].astype(jnp.bfloat16)
        rdma_b[d] = exchange(rs_send_ref, ofs, rs_recv_ref, ofs, 2,
                             d - 1, rs_sems.at[3 + d - 1], my ^ (d << 2))
    for d in (1, 2, 3):
        rdma_b[d].wait_recv()
    fs = rb + qb * 2
    acc_ref[pl.ds(fs, 2)] = (
        acc_ref[pl.ds(fs, 2)]
        + rs_recv_ref[pl.ds(24, 2)].astype(jnp.float32)
        + rs_recv_ref[pl.ds(26, 2)].astype(jnp.float32)
        + rs_recv_ref[pl.ds(28, 2)].astype(jnp.float32))
    for d in (1, 2, 3):
        rdma_b[d].wait_send()

    rs_send_ref[pl.ds(30, 2)] = acc_ref[pl.ds(fs, 2)].astype(jnp.bfloat16)
    are = exchange(rs_send_ref, 30, rs_recv_ref, 30, 2,
                   0, rs_sems.at[6], my ^ 16)
    are.wait_recv()
    acc_ref[pl.ds(fs, 2)] = (
        acc_ref[pl.ds(fs, 2)]
        + rs_recv_ref[pl.ds(30, 2)].astype(jnp.float32))
    are.wait_send()

    ag_send_ref[pl.ds(0, 2)] = acc_ref[pl.ds(fs, 2)].astype(jnp.bfloat16)
    rdma_gb = {}
    for d in (1, 2, 3):
        rdma_gb[d] = exchange(ag_send_ref, 0, ag_recv_ref, (d - 1) * 2, 2,
                              d - 1, ag_sems.at[d - 1], my ^ (d << 2))
    for d in (1, 2, 3):
        rdma_gb[d].wait_recv()
        acc_ref[pl.ds(rb + (qb ^ d) * 2, 2)] = ag_recv_ref[
            pl.ds((d - 1) * 2, 2)].astype(jnp.float32)
    for d in (1, 2, 3):
        rdma_gb[d].wait_send()

    ag_send_ref[pl.ds(2, 8)] = acc_ref[pl.ds(rb, 8)].astype(jnp.bfloat16)
    rdma_ga = {}
    for d in (1, 2, 3):
        rdma_ga[d] = exchange(ag_send_ref, 2, ag_recv_ref, 6 + (d - 1) * 8, 8,
                              d - 1, ag_sems.at[3 + d - 1], my ^ d)

    for qv in range(4):
        @pl.when(qa == qv)
        def _(qv=qv):
            r0 = (qv & 1) * 128
            out_ref[qv >> 1, r0:r0 + 128] = acc_ref[
                qv * 8:(qv + 1) * 8].reshape(128, D_MODEL)

    for d in (1, 3, 2):
        rdma_ga[d].wait_recv()
    for qv in range(4):
        @pl.when(qa == qv)
        def _(qv=qv):
            for d in (1, 2, 3):
                c = qv ^ d
                r0 = (c & 1) * 128
                out_ref[c >> 1, r0:r0 + 128] = ag_recv_ref[
                    6 + (d - 1) * 8:6 + d * 8].astype(jnp.float32).reshape(
                        128, D_MODEL)
    for d in (1, 2, 3):
        rdma_ga[d].wait_send()


def kernel(x, Wq, Wo, K_ext, V_ext):
    i = lax.axis_index("i")
    k_loc = lax.dynamic_slice_in_dim(K_ext, KV_LOC * i, KV_LOC, axis=2)
    v_loc = lax.dynamic_slice_in_dim(V_ext, KV_LOC * i, KV_LOC, axis=2)
    k_loc = k_loc.transpose(0, 2, 1, 3)
    v_loc = v_loc.transpose(0, 2, 1, 3)

    return pl.pallas_call(
        _body,
        out_shape=jax.ShapeDtypeStruct((B, SQ, D_MODEL), jnp.float32),
        in_specs=[pl.BlockSpec(memory_space=pltpu.VMEM)] * 5,
        out_specs=pl.BlockSpec(memory_space=pltpu.VMEM),
        scratch_shapes=[
            pltpu.VMEM((N_CHUNKS, CHUNK_ROWS, D_MODEL), jnp.float32),
            pltpu.VMEM((32, CHUNK_ROWS, D_MODEL), jnp.bfloat16),
            pltpu.VMEM((32, CHUNK_ROWS, D_MODEL), jnp.bfloat16),
            pltpu.VMEM((10, CHUNK_ROWS, D_MODEL), jnp.bfloat16),
            pltpu.VMEM((30, CHUNK_ROWS, D_MODEL), jnp.bfloat16),
            pltpu.SemaphoreType.DMA((3,)),
            pltpu.SemaphoreType.DMA((7,)),
            pltpu.SemaphoreType.DMA((6,)),
        ],
        compiler_params=pltpu.CompilerParams(collective_id=0),
    )(x, Wq, Wo, k_loc, v_loc)


# device time: 37599 ns/iter; 2.5295x vs baseline; 1.0111x over previous
import jax
import jax.numpy as jnp
from jax import lax
from jax.experimental import pallas as pl
from jax.experimental.pallas import tpu as pltpu

N_DEV = 32
B = 2
SQ = 256
D_MODEL = 768
H_LOC = 8
GQA_GROUP = 4
KV_LOC = H_LOC // GQA_GROUP
DH = 64
SKV = 512
ROWS = B * SQ
N_CHUNKS = N_DEV
CHUNK_ROWS = ROWS // N_CHUNKS
CHUNKS_PER_B = SQ // CHUNK_ROWS

BARRIER_MASKS = (1, 2, 3, 4, 8, 12, 16)


def _body(x_ref, wq_ref, wo_ref, k_ref, v_ref, out_ref,
          acc_ref, rs_send_ref, rs_recv_ref, ag_send_ref, ag_recv_ref,
          send_sems, rs_sems, ag_sems):
    my = lax.axis_index("i")
    qa = my & 3
    qb = (my >> 2) & 3
    b_keep = (my >> 1) & 1

    barrier = pltpu.get_barrier_semaphore()
    for msk in BARRIER_MASKS:
        pl.semaphore_signal(barrier, inc=1, device_id=(my ^ msk,),
                            device_id_type=pl.DeviceIdType.MESH)
    pl.semaphore_wait(barrier, len(BARRIER_MASKS))

    def compute_partial(b):
        qmat = jnp.dot(x_ref[b], wq_ref[...],
                       preferred_element_type=jnp.float32)
        outs = []
        for h in range(H_LOC):
            q = qmat[:, h * DH:(h + 1) * DH]
            k = k_ref[b, h // GQA_GROUP]
            v = v_ref[b, h // GQA_GROUP]
            s = jnp.dot(q, k.T, preferred_element_type=jnp.float32) * 0.125
            m = jnp.max(s, axis=1, keepdims=True)
            p = jnp.exp(s - m)
            l = jnp.sum(p, axis=1, keepdims=True)
            outs.append(jnp.dot(p, v, preferred_element_type=jnp.float32) / l)
        o = jnp.concatenate(outs, axis=1)
        pb = jnp.dot(o, wo_ref[...],
                     preferred_element_type=jnp.float32)
        acc_ref[pl.ds(b * CHUNKS_PER_B, CHUNKS_PER_B)] = pb.reshape(
            CHUNKS_PER_B, CHUNK_ROWS, D_MODEL)

    def exchange(send_buf, send_ofs, recv_buf, recv_ofs, n, sem_i, recv_sem,
                 partner):
        rdma = pltpu.make_async_remote_copy(
            src_ref=send_buf.at[pl.ds(send_ofs, n)],
            dst_ref=recv_buf.at[pl.ds(recv_ofs, n)],
            send_sem=send_sems.at[sem_i],
            recv_sem=recv_sem,
            device_id=(partner,),
            device_id_type=pl.DeviceIdType.MESH,
        )
        rdma.start()
        return rdma

    @pl.when(b_keep == 0)
    def _():
        compute_partial(1)

    @pl.when(b_keep == 1)
    def _():
        compute_partial(0)

    rdma_a = {}
    for d in (2, 3):
        ofs = (d - 1) * 8
        rs_send_ref[pl.ds(ofs, 8)] = acc_ref[
            pl.ds((qa ^ d) * 8, 8)].astype(jnp.bfloat16)
        rdma_a[d] = exchange(rs_send_ref, ofs, rs_recv_ref, ofs, 8,
                             d - 1, rs_sems.at[d - 1], my ^ d)

    @pl.when(b_keep == 0)
    def _():
        compute_partial(0)

    @pl.when(b_keep == 1)
    def _():
        compute_partial(1)

    rs_send_ref[pl.ds(0, 8)] = acc_ref[pl.ds((qa ^ 1) * 8, 8)].astype(
        jnp.bfloat16)
    rdma_a[1] = exchange(rs_send_ref, 0, rs_recv_ref, 0, 8,
                         0, rs_sems.at[0], my ^ 1)

    for d in (1, 3, 2):
        rdma_a[d].wait_recv()
    acc_ref[pl.ds(qa * 8, 8)] = (
        acc_ref[pl.ds(qa * 8, 8)]
        + rs_recv_ref[pl.ds(0, 8)].astype(jnp.float32)
        + rs_recv_ref[pl.ds(8, 8)].astype(jnp.float32)
        + rs_recv_ref[pl.ds(16, 8)].astype(jnp.float32))
    for d in (1, 2, 3):
        rdma_a[d].wait_send()

    rb = qa * 8
    rdma_b = {}
    for d in (1, 2, 3):
        ofs = 24 + (d - 1) * 2
        rs_send_ref[pl.ds(ofs, 2)] = acc_ref[
            pl.ds(rb + (qb ^ d) * 2, 2)].astype(jnp.bfloat16)
        rdma_b[d] = exchange(rs_send_ref, ofs, rs_recv_ref, ofs, 2,
                             d - 1, rs_sems.at[3 + d - 1], my ^ (d << 2))
    for d in (1, 2, 3):
        rdma_b[d].wait_recv()
    fs = rb + qb * 2
    acc_ref[pl.ds(fs, 2)] = (
        acc_ref[pl.ds(fs, 2)]
        + rs_recv_ref[pl.ds(24, 2)].astype(jnp.float32)
        + rs_recv_ref[pl.ds(26, 2)].astype(jnp.float32)
        + rs_recv_ref[pl.ds(28, 2)].astype(jnp.float32))
    for d in (1, 2, 3):
        rdma_b[d].wait_send()

    rs_send_ref[pl.ds(30, 2)] = acc_ref[pl.ds(fs, 2)].astype(jnp.bfloat16)
    are = exchange(rs_send_ref, 30, rs_recv_ref, 30, 2,
                   0, rs_sems.at[6], my ^ 16)
    are.wait_recv()
    acc_ref[pl.ds(fs, 2)] = (
        acc_ref[pl.ds(fs, 2)]
        + rs_recv_ref[pl.ds(30, 2)].astype(jnp.float32))
    are.wait_send()

    ag_send_ref[pl.ds(0, 2)] = acc_ref[pl.ds(fs, 2)].astype(jnp.bfloat16)
    rdma_gb = {}
    for d in (1, 2, 3):
        rdma_gb[d] = exchange(ag_send_ref, 0, ag_recv_ref, (d - 1) * 2, 2,
                              d - 1, ag_sems.at[d - 1], my ^ (d << 2))
    for d in (1, 2, 3):
        rdma_gb[d].wait_recv()
        acc_ref[pl.ds(rb + (qb ^ d) * 2, 2)] = ag_recv_ref[
            pl.ds((d - 1) * 2, 2)].astype(jnp.float32)
    for d in (1, 2, 3):
        rdma_gb[d].wait_send()

    ag_send_ref[pl.ds(2, 8)] = acc_ref[pl.ds(rb, 8)].astype(jnp.bfloat16)
    rdma_ga = {}
    for d in (1, 2, 3):
        rdma_ga[d] = exchange(ag_send_ref, 2, ag_recv_ref, 6 + (d - 1) * 8, 8,
                              d - 1, ag_sems.at[3 + d - 1], my ^ d)
    for d in (1, 2, 3):
        rdma_ga[d].wait_recv()
        acc_ref[pl.ds((qa ^ d) * 8, 8)] = ag_recv_ref[
            pl.ds(6 + (d - 1) * 8, 8)].astype(jnp.float32)
    for d in (1, 2, 3):
        rdma_ga[d].wait_send()

    for b in range(B):
        out_ref[b] = acc_ref[b * CHUNKS_PER_B:(b + 1) * CHUNKS_PER_B].reshape(
            SQ, D_MODEL)


def kernel(x, Wq, Wo, K_ext, V_ext):
    i = lax.axis_index("i")
    k_loc = lax.dynamic_slice_in_dim(K_ext, KV_LOC * i, KV_LOC, axis=2)
    v_loc = lax.dynamic_slice_in_dim(V_ext, KV_LOC * i, KV_LOC, axis=2)
    k_loc = k_loc.transpose(0, 2, 1, 3)
    v_loc = v_loc.transpose(0, 2, 1, 3)

    return pl.pallas_call(
        _body,
        out_shape=jax.ShapeDtypeStruct((B, SQ, D_MODEL), jnp.float32),
        in_specs=[pl.BlockSpec(memory_space=pltpu.VMEM)] * 5,
        out_specs=pl.BlockSpec(memory_space=pltpu.VMEM),
        scratch_shapes=[
            pltpu.VMEM((N_CHUNKS, CHUNK_ROWS, D_MODEL), jnp.float32),
            pltpu.VMEM((32, CHUNK_ROWS, D_MODEL), jnp.bfloat16),
            pltpu.VMEM((32, CHUNK_ROWS, D_MODEL), jnp.bfloat16),
            pltpu.VMEM((10, CHUNK_ROWS, D_MODEL), jnp.bfloat16),
            pltpu.VMEM((30, CHUNK_ROWS, D_MODEL), jnp.bfloat16),
            pltpu.SemaphoreType.DMA((3,)),
            pltpu.SemaphoreType.DMA((7,)),
            pltpu.SemaphoreType.DMA((6,)),
        ],
        compiler_params=pltpu.CompilerParams(collective_id=0),
    )(x, Wq, Wo, k_loc, v_loc)
